# Initial kernel scaffold; baseline (speedup 1.0000x reference)
import jax
import jax.numpy as jnp
from jax.experimental import pallas as pl


def kernel(positions, particle_weights, cell):
    raise NotImplementedError("write your pallas kernel here")


# speedup vs baseline: 27.6455x; 27.6455x over previous
import functools

import jax
import jax.numpy as jnp
from jax import lax
from jax.experimental import pallas as pl
from jax.experimental.pallas import tpu as pltpu
from jax.experimental.pallas import tpu_sc as plsc

N = 100000
NX = NY = NZ = 128
C = 4
R = 16
NPT = 6256
PADN = 100352
NSLOT = 6272
NPAD = 101632
TRASH = NPAD - 256
CHUNK = 512
TAIL = 112
GC = 256

_i32 = jnp.int32
_f32 = jnp.float32


def _p3m_weights(x):
    x2 = x * x
    x3 = x2 * x
    w0 = (1.0 - 6.0 * x + 12.0 * x2 - 8.0 * x3) * (1.0 / 48.0)
    w1 = (23.0 - 30.0 * x - 12.0 * x2 + 24.0 * x3) * (1.0 / 48.0)
    w2 = (23.0 + 30.0 * x - 12.0 * x2 - 24.0 * x3) * (1.0 / 48.0)
    w3 = (1.0 + 6.0 * x + 12.0 * x2 + 8.0 * x3) * (1.0 / 48.0)
    return w0, w1, w2, w3


def _body(pos_ref, pw_ref, out_ref, rec_ref, srt_ref,
          pxb, pyb, pzb, pwc, recbuf, ixbuf, hist, ah, cur, posb, idb,
          plane, recchunk, idchunk, wbuf, ybuf, zbuf, z512,
          shh, obuf, tbuf, sem):
    cc_ = lax.axis_index("c")
    ss_ = lax.axis_index("s")
    iota = lax.iota(_i32, 16)
    izero = jnp.zeros((16,), _i32)
    fzero = jnp.zeros((16,), _f32)
    ione = jnp.ones((16,), _i32)

    gstart = ss_ * NPT
    np_t = jnp.where(ss_ == 15, N - 15 * NPT, NPT)


    @pl.loop(0, 128)
    def _zh(i):
        hist[pl.ds(i * 16, 16)] = izero

    @pl.loop(0, 32)
    def _zz(i):
        z512[pl.ds(i * 16, 16)] = izero

    def prep_chunk(ck, size, nv):
        gbase = pl.multiple_of(gstart + ck * CHUNK, 8)
        pltpu.sync_copy(pos_ref.at[0, pl.ds(gbase, size)], pxb.at[pl.ds(0, size)])
        pltpu.sync_copy(pos_ref.at[1, pl.ds(gbase, size)], pyb.at[pl.ds(0, size)])
        pltpu.sync_copy(pos_ref.at[2, pl.ds(gbase, size)], pzb.at[pl.ds(0, size)])
        pltpu.sync_copy(pw_ref.at[pl.ds(gbase, size)], pwc.at[pl.ds(0, size)])

        @pl.loop(0, nv)
        def _v(v):
            base = v * 16
            rows = base + iota
            rx = pxb[pl.ds(base, 16)]
            ry = pyb[pl.ds(base, 16)]
            rz = pzb[pl.ds(base, 16)]
            ixr = rx.astype(_i32)
            iyr = ry.astype(_i32)
            izr = rz.astype(_i32)
            flx = ixr.astype(_f32)
            fly = iyr.astype(_f32)
            flz = izr.astype(_f32)
            ix = ixr & 127
            iy = iyr & 127
            iz = izr & 127
            wx0, wx1, wx2, _ = _p3m_weights(rx - flx - 0.5)
            wy0, wy1, wy2, wy3 = _p3m_weights(ry - fly - 0.5)
            wz0, wz1, wz2, wz3 = _p3m_weights(rz - flz - 0.5)

            slot = ck * CHUNK + base
            valid = (slot + iota) < np_t
            plsc.addupdate_scatter(hist, [ix * 16 + iota], ione, mask=valid)
            ixbuf[pl.ds(slot, 16)] = ix

            def putf(col, val):
                plsc.store_scatter(recbuf, [rows, jnp.full((16,), col, _i32)], val)

            putf(0, plsc.bitcast(iy * 128 + iz, _f32))
            putf(1, wx0)
            putf(2, wx1)
            putf(3, wx2)
            putf(4, wy0)
            putf(5, wy1)
            putf(6, wy2)
            putf(7, wy3)
            putf(8, wz0)
            putf(9, wz1)
            putf(10, wz2)
            putf(11, wz3)
            for q in range(4):
                pv = plsc.load_gather(pwc, [rows, jnp.full((16,), q, _i32)])
                putf(12 + q, pv)

        pltpu.sync_copy(recbuf.at[pl.ds(0, size)],
                        rec_ref.at[cc_, pl.ds(gbase, size)])

    @pl.loop(0, 12)
    def _pc(ck):
        prep_chunk(ck, CHUNK, 32)

    prep_chunk(12, TAIL, 7)

    pltpu.sync_copy(hist, shh.at[ss_])

    pslice = ss_ * (NPAD // 16)

    @pl.loop(0, 12)
    def _pf(k):
        pltpu.sync_copy(z512, srt_ref.at[cc_, pl.ds(
            pl.multiple_of(pslice + k * CHUNK, 8), CHUNK)])

    pltpu.sync_copy(z512.at[pl.ds(0, NPAD // 16 - 12 * CHUNK)],
                    srt_ref.at[cc_, pl.ds(
                        pl.multiple_of(pslice + 12 * CHUNK, 8),
                        NPAD // 16 - 12 * CHUNK)])

    plsc.subcore_barrier()

    acc = jnp.int32(0)
    for h in range(2):
        pltpu.sync_copy(shh.at[:, pl.ds(h * 1024, 1024)], ah)

        def _bb(bb, a):
            b = h * 64 + bb
            run = jnp.int32(0)
            for t in range(16):
                v = ah[t, pl.ds(bb * 16, 16)]
                excl = plsc.cumsum(v) - v
                tot = jnp.sum(v)

                @pl.when(t == ss_)
                def _st():
                    cur[pl.ds(b * 16, 16)] = excl + (a + run)

                run = run + tot
            obuf[b] = a
            tbuf[b] = run
            return (a + run + 7) & ~jnp.int32(7)

        acc = pl.loop(0, 64, init_carry=acc)(_bb)

    def _rb(vv, carry):
        pcidx, ppos = carry
        base = vv * 16
        ixv = ixbuf[pl.ds(base, 16)] & 127
        valid = (base + iota) < np_t
        cidx = ixv * 16 + iota
        raw = plsc.load_gather(cur, [cidx])
        pos = jnp.where(cidx == pcidx, ppos + 1, raw)
        plsc.store_scatter(cur, [cidx], pos + 1, mask=valid)
        posv = jnp.where(valid, pos, TRASH + iota)
        idv = jnp.where(valid, gstart + base + iota, 0)
        r = vv >> 3
        col = (vv & 7) * 16 + iota
        plsc.store_scatter(posb, [jnp.full((16,), r, _i32), col], posv)
        plsc.store_scatter(idb, [jnp.full((16,), r, _i32), col], idv)
        return (jnp.where(valid, cidx, -1), pos)

    pl.loop(0, NSLOT // 16,
            init_carry=(jnp.full((16,), -1, _i32),
                        jnp.zeros((16,), _i32)))(_rb)

    @pl.loop(0, 49)
    def _sb(r):
        pltpu.sync_copy(idb.at[r], srt_ref.at[cc_].at[posb.at[r]])

    @pl.loop(0, 49)
    def _sb2(r):
        pltpu.sync_copy(srt_ref.at[cc_].at[posb.at[r]], idb.at[r])

    plsc.subcore_barrier()
    @pl.loop(0, 12)
    def _fl(k):
        pltpu.sync_copy(srt_ref.at[cc_, pl.ds(
            pl.multiple_of(pslice + k * CHUNK, 8), CHUNK)], z512)

    pltpu.sync_copy(srt_ref.at[cc_, pl.ds(
        pl.multiple_of(pslice + 12 * CHUNK, 8),
        NPAD // 16 - 12 * CHUNK)],
        z512.at[pl.ds(0, NPAD // 16 - 12 * CHUNK)])

    plsc.subcore_barrier()

    wid = ss_ * 2 + cc_

    @pl.loop(0, 16)
    def _zw(i):
        plsc.store_scatter(wbuf, [iota, jnp.broadcast_to(i, (16,))], fzero)
        plsc.store_scatter(ybuf, [iota, jnp.broadcast_to(i, (16,))], izero)
        plsc.store_scatter(zbuf, [iota, jnp.broadcast_to(i, (16,))], izero)

    def do_chunk(st, st1, cnt0, tb, kx, kxcol):
        pltpu.sync_copy(srt_ref.at[cc_, pl.ds(st, 128)], idchunk.at[0])
        pltpu.sync_copy(srt_ref.at[cc_, pl.ds(st1, 128)], idchunk.at[1])
        pltpu.sync_copy(rec_ref.at[cc_].at[idchunk.at[0]],
                        recchunk.at[pl.ds(0, 128)])
        pltpu.sync_copy(rec_ref.at[cc_].at[idchunk.at[1]],
                        recchunk.at[pl.ds(128, 128)])

        dym = (iota >> 2) - 1
        dzm = (iota & 3) - 1
        dyv = iota >> 2
        dzv = iota & 3

        @pl.loop(0, 16)
        def _v(v):
            rows = v * 16 + iota

            def g(col):
                return plsc.load_gather(
                    recchunk, [rows, jnp.full((16,), col, _i32)])

            yz = plsc.bitcast(g(0), _i32)
            iy = yz >> 7
            iz = yz & 127
            w0, w1, w2 = g(1), g(2), g(3)
            wa = plsc.load_gather(
                recchunk, [rows, jnp.full((16,), kxcol, _i32)])
            wxk = jnp.where(kx == 3, 1.0 - w0 - w1 - w2, wa)
            wy = [g(4 + d) for d in range(4)]
            wz = [g(8 + d) for d in range(4)]
            pwv = [g(12 + q) for q in range(4)]
            cnt = cnt0 + v * 16
            _dn = lax.GatherDimensionNumbers(
                offset_dims=(), collapsed_slice_dims=(0,),
                start_index_map=(0,))
            for q in range(16):
                fq = jnp.full((16, 1), q, _i32)

                def tk(x):
                    return lax.gather(
                        x, fq, dimension_numbers=_dn, slice_sizes=(1,),
                        mode=lax.GatherScatterMode.PROMISE_IN_BOUNDS)

                yrow = (tk(iy) + dym) & 127
                zrow = (tk(iz) + dzm) & 127
                wyv = jnp.where(
                    dyv == 0, tk(wy[0]),
                    jnp.where(dyv == 1, tk(wy[1]),
                              jnp.where(dyv == 2, tk(wy[2]), tk(wy[3]))))
                wzv = jnp.where(
                    dzv == 0, tk(wz[0]),
                    jnp.where(dzv == 1, tk(wz[1]),
                              jnp.where(dzv == 2, tk(wz[2]), tk(wz[3]))))
                wvq = tk(wxk) * wyv * wzv
                vm = jnp.full((16,), (cnt + q) < tb)
                for ch in range(4):
                    plsc.addupdate_scatter(
                        plane,
                        [jnp.full((16,), ch, _i32), yrow, zrow],
                        wvq * tk(pwv[ch]), mask=vm)

    do_chunk(jnp.int32(0), jnp.int32(128), jnp.int32(0), jnp.int32(GC),
             jnp.int32(0), jnp.int32(1))

    @pl.loop(0, 4)
    def _pp(jp):
        p = wid * 4 + jp

        @pl.loop(0, 128)
        def _zp(i):
            for ch in range(4):
                for j in range(8):
                    plane[ch, i, pl.ds(j * 16, 16)] = fzero

        plsc.subcore_barrier()

        @pl.loop(0, 4)
        def _bk(jb):
            b = (p - 2 + jb) & 127
            kx = 3 - jb
            ab = obuf[b]
            tb = tbuf[b]
            nch = (tb + (GC - 1)) // GC
            kxcol = 1 + jnp.minimum(kx, 2)

            @pl.loop(0, nch)
            def _ck(k):
                do_chunk(pl.multiple_of(ab + k * GC, 8),
                         pl.multiple_of(ab + k * GC + 128, 8),
                         k * GC, tb, kx, kxcol)

        for ch in range(4):
            pltpu.sync_copy(plane.at[ch], out_ref.at[ch, p])


def kernel(positions, particle_weights, cell):
    ns = jnp.array([NX, NY, NZ], dtype=positions.dtype)
    inv_cell = jnp.linalg.inv(cell)
    rel = ns * jnp.matmul(positions, inv_cell)

    pos_t = jnp.pad(rel.T, ((0, 0), (0, PADN - N)))
    pw_p = jnp.pad(particle_weights, ((0, PADN - N), (0, 0)))

    mesh = plsc.VectorSubcoreMesh(core_axis_name="c", subcore_axis_name="s")
    f = pl.kernel(
        _body,
        out_type=[
            jax.ShapeDtypeStruct((C, NX, NY, NZ), _f32),
            jax.ShapeDtypeStruct((2, PADN, R), _f32),
            jax.ShapeDtypeStruct((2, NPAD), _i32),
        ],
        mesh=mesh,
        scratch_types=[
            pltpu.VMEM((CHUNK,), _f32),
            pltpu.VMEM((CHUNK,), _f32),
            pltpu.VMEM((CHUNK,), _f32),
            pltpu.VMEM((CHUNK, 4), _f32),
            pltpu.VMEM((CHUNK, R), _f32),
            pltpu.VMEM((NSLOT,), _i32),
            pltpu.VMEM((2048,), _i32),
            pltpu.VMEM((16, 1024), _i32),
            pltpu.VMEM((2048,), _i32),
            pltpu.VMEM((49, 128), _i32),
            pltpu.VMEM((49, 128), _i32),
            pltpu.VMEM((C, NY, NZ), _f32),
            pltpu.VMEM((GC, R), _f32),
            pltpu.VMEM((2, 128), _i32),
            pltpu.VMEM((16, 16), _f32),
            pltpu.VMEM((16, 16), _i32),
            pltpu.VMEM((16, 16), _i32),
            pltpu.VMEM((CHUNK,), _i32),
            pltpu.VMEM_SHARED((16, 2048), _i32),
            pltpu.SMEM((136,), _i32),
            pltpu.SMEM((136,), _i32),
            pltpu.SemaphoreType.DMA,
        ],
        compiler_params=pltpu.CompilerParams(
            use_tc_tiling_on_sc=False, needs_layout_passes=False),
    )
    rho, _, _ = f(pos_t, pw_p)
    return rho


# speedup vs baseline: 28.8406x; 1.0432x over previous
import functools

import jax
import jax.numpy as jnp
from jax import lax
from jax.experimental import pallas as pl
from jax.experimental.pallas import tpu as pltpu
from jax.experimental.pallas import tpu_sc as plsc

N = 100000
NX = NY = NZ = 128
C = 4
R = 16
NPT = 6256
PADN = 100352
NSLOT = 6272
NPAD = 101632
TRASH = NPAD - 256
CHUNK = 512
TAIL = 112
GC = 256

_i32 = jnp.int32
_f32 = jnp.float32


def _p3m_weights(x):
    x2 = x * x
    x3 = x2 * x
    w0 = (1.0 - 6.0 * x + 12.0 * x2 - 8.0 * x3) * (1.0 / 48.0)
    w1 = (23.0 - 30.0 * x - 12.0 * x2 + 24.0 * x3) * (1.0 / 48.0)
    w2 = (23.0 + 30.0 * x - 12.0 * x2 - 24.0 * x3) * (1.0 / 48.0)
    w3 = (1.0 + 6.0 * x + 12.0 * x2 + 8.0 * x3) * (1.0 / 48.0)
    return w0, w1, w2, w3


def _body(pos_ref, pw_ref, out_ref, rec_ref, srt_ref,
          pxb, pyb, pzb, pwc, recbuf, ixbuf, hist, ah, cur, posb, idb,
          plane, recchunk, idchunk, wbuf, ybuf, zbuf, z512,
          shh, obuf, tbuf, sem):
    cc_ = lax.axis_index("c")
    ss_ = lax.axis_index("s")
    iota = lax.iota(_i32, 16)
    izero = jnp.zeros((16,), _i32)
    fzero = jnp.zeros((16,), _f32)
    ione = jnp.ones((16,), _i32)

    gstart = ss_ * NPT
    np_t = jnp.where(ss_ == 15, N - 15 * NPT, NPT)


    @pl.loop(0, 128)
    def _zh(i):
        hist[pl.ds(i * 16, 16)] = izero

    @pl.loop(0, 32)
    def _zz(i):
        z512[pl.ds(i * 16, 16)] = izero

    def prep_chunk(ck, size, nv):
        gbase = pl.multiple_of(gstart + ck * CHUNK, 8)
        pltpu.sync_copy(pos_ref.at[0, pl.ds(gbase, size)], pxb.at[pl.ds(0, size)])
        pltpu.sync_copy(pos_ref.at[1, pl.ds(gbase, size)], pyb.at[pl.ds(0, size)])
        pltpu.sync_copy(pos_ref.at[2, pl.ds(gbase, size)], pzb.at[pl.ds(0, size)])
        pltpu.sync_copy(pw_ref.at[pl.ds(gbase, size)], pwc.at[pl.ds(0, size)])

        @pl.loop(0, nv)
        def _v(v):
            base = v * 16
            rows = base + iota
            rx = pxb[pl.ds(base, 16)]
            ry = pyb[pl.ds(base, 16)]
            rz = pzb[pl.ds(base, 16)]
            ixr = rx.astype(_i32)
            iyr = ry.astype(_i32)
            izr = rz.astype(_i32)
            flx = ixr.astype(_f32)
            fly = iyr.astype(_f32)
            flz = izr.astype(_f32)
            ix = ixr & 127
            iy = iyr & 127
            iz = izr & 127
            wx0, wx1, wx2, _ = _p3m_weights(rx - flx - 0.5)
            wy0, wy1, wy2, wy3 = _p3m_weights(ry - fly - 0.5)
            wz0, wz1, wz2, wz3 = _p3m_weights(rz - flz - 0.5)

            slot = ck * CHUNK + base
            valid = (slot + iota) < np_t
            plsc.addupdate_scatter(hist, [ix * 16 + iota], ione, mask=valid)
            ixbuf[pl.ds(slot, 16)] = ix

            def putf(col, val):
                plsc.store_scatter(recbuf, [rows, jnp.full((16,), col, _i32)], val)

            putf(0, plsc.bitcast(iy * 128 + iz, _f32))
            putf(1, wx0)
            putf(2, wx1)
            putf(3, wx2)
            putf(4, wy0)
            putf(5, wy1)
            putf(6, wy2)
            putf(7, wy3)
            putf(8, wz0)
            putf(9, wz1)
            putf(10, wz2)
            putf(11, wz3)
            for q in range(4):
                pv = plsc.load_gather(pwc, [rows, jnp.full((16,), q, _i32)])
                putf(12 + q, pv)

        pltpu.sync_copy(recbuf.at[pl.ds(0, size)],
                        rec_ref.at[cc_, pl.ds(gbase, size)])

    @pl.loop(0, 12)
    def _pc(ck):
        prep_chunk(ck, CHUNK, 32)

    prep_chunk(12, TAIL, 7)

    pltpu.sync_copy(hist, shh.at[ss_])

    pslice = ss_ * (NPAD // 16)

    @pl.loop(0, 12)
    def _pf(k):
        pltpu.sync_copy(z512, srt_ref.at[cc_, pl.ds(
            pl.multiple_of(pslice + k * CHUNK, 8), CHUNK)])

    pltpu.sync_copy(z512.at[pl.ds(0, NPAD // 16 - 12 * CHUNK)],
                    srt_ref.at[cc_, pl.ds(
                        pl.multiple_of(pslice + 12 * CHUNK, 8),
                        NPAD // 16 - 12 * CHUNK)])

    plsc.subcore_barrier()

    acc = jnp.int32(0)
    for h in range(2):
        pltpu.sync_copy(shh.at[:, pl.ds(h * 1024, 1024)], ah)

        def _bb(bb, a):
            b = h * 64 + bb
            run = jnp.int32(0)
            for t in range(16):
                v = ah[t, pl.ds(bb * 16, 16)]
                excl = plsc.cumsum(v) - v
                tot = jnp.sum(v)

                @pl.when(t == ss_)
                def _st():
                    cur[pl.ds(b * 16, 16)] = excl + (a + run)

                run = run + tot
            obuf[b] = a
            tbuf[b] = run
            return (a + run + 7) & ~jnp.int32(7)

        acc = pl.loop(0, 64, init_carry=acc)(_bb)

    def _rb(vv, carry):
        pcidx, ppos = carry
        base = vv * 16
        ixv = ixbuf[pl.ds(base, 16)] & 127
        valid = (base + iota) < np_t
        cidx = ixv * 16 + iota
        raw = plsc.load_gather(cur, [cidx])
        pos = jnp.where(cidx == pcidx, ppos + 1, raw)
        plsc.store_scatter(cur, [cidx], pos + 1, mask=valid)
        posv = jnp.where(valid, pos, TRASH + iota)
        idv = jnp.where(valid, gstart + base + iota, 0)
        r = vv >> 3
        col = (vv & 7) * 16 + iota
        plsc.store_scatter(posb, [jnp.full((16,), r, _i32), col], posv)
        plsc.store_scatter(idb, [jnp.full((16,), r, _i32), col], idv)
        return (jnp.where(valid, cidx, -1), pos)

    pl.loop(0, NSLOT // 16,
            init_carry=(jnp.full((16,), -1, _i32),
                        jnp.zeros((16,), _i32)))(_rb)

    @pl.loop(0, 49)
    def _sb(r):
        pltpu.sync_copy(idb.at[r], srt_ref.at[cc_].at[posb.at[r]])

    @pl.loop(0, 49)
    def _sb2(r):
        pltpu.sync_copy(srt_ref.at[cc_].at[posb.at[r]], idb.at[r])

    plsc.subcore_barrier()
    @pl.loop(0, 12)
    def _fl(k):
        pltpu.sync_copy(srt_ref.at[cc_, pl.ds(
            pl.multiple_of(pslice + k * CHUNK, 8), CHUNK)], z512)

    pltpu.sync_copy(srt_ref.at[cc_, pl.ds(
        pl.multiple_of(pslice + 12 * CHUNK, 8),
        NPAD // 16 - 12 * CHUNK)],
        z512.at[pl.ds(0, NPAD // 16 - 12 * CHUNK)])

    plsc.subcore_barrier()

    wid = ss_ * 2 + cc_

    @pl.loop(0, 16)
    def _zw(i):
        plsc.store_scatter(wbuf, [iota, jnp.broadcast_to(i, (16,))], fzero)
        plsc.store_scatter(ybuf, [iota, jnp.broadcast_to(i, (16,))], izero)
        plsc.store_scatter(zbuf, [iota, jnp.broadcast_to(i, (16,))], izero)

    def do_chunk(st, st1, cnt0, tb, kx, kxcol):
        pltpu.sync_copy(srt_ref.at[cc_, pl.ds(st, 128)], idchunk.at[0])
        pltpu.sync_copy(srt_ref.at[cc_, pl.ds(st1, 128)], idchunk.at[1])
        d0 = pltpu.async_copy(rec_ref.at[cc_].at[idchunk.at[0]],
                              recchunk.at[pl.ds(0, 128)], sem)
        d1 = pltpu.async_copy(rec_ref.at[cc_].at[idchunk.at[1]],
                              recchunk.at[pl.ds(128, 128)], sem)
        d0.wait()
        d1.wait()

        dym = (iota >> 2) - 1
        dzm = (iota & 3) - 1
        dyv = iota >> 2
        dzv = iota & 3

        @pl.loop(0, 16)
        def _v(v):
            rows = v * 16 + iota

            def g(col):
                return plsc.load_gather(
                    recchunk, [rows, jnp.full((16,), col, _i32)])

            yz = plsc.bitcast(g(0), _i32)
            iy = yz >> 7
            iz = yz & 127
            w0, w1, w2 = g(1), g(2), g(3)
            wa = plsc.load_gather(
                recchunk, [rows, jnp.full((16,), kxcol, _i32)])
            wxk = jnp.where(kx == 3, 1.0 - w0 - w1 - w2, wa)
            wy = [g(4 + d) for d in range(4)]
            wz = [g(8 + d) for d in range(4)]
            pwv = [g(12 + q) for q in range(4)]
            cnt = cnt0 + v * 16
            _dn = lax.GatherDimensionNumbers(
                offset_dims=(), collapsed_slice_dims=(0,),
                start_index_map=(0,))
            for q in range(16):
                fq = jnp.full((16, 1), q, _i32)

                def tk(x):
                    return lax.gather(
                        x, fq, dimension_numbers=_dn, slice_sizes=(1,),
                        mode=lax.GatherScatterMode.PROMISE_IN_BOUNDS)

                yrow = (tk(iy) + dym) & 127
                zrow = (tk(iz) + dzm) & 127
                wyv = jnp.where(
                    dyv == 0, tk(wy[0]),
                    jnp.where(dyv == 1, tk(wy[1]),
                              jnp.where(dyv == 2, tk(wy[2]), tk(wy[3]))))
                wzv = jnp.where(
                    dzv == 0, tk(wz[0]),
                    jnp.where(dzv == 1, tk(wz[1]),
                              jnp.where(dzv == 2, tk(wz[2]), tk(wz[3]))))
                wvq = tk(wxk) * wyv * wzv
                vm = jnp.full((16,), (cnt + q) < tb)
                for ch in range(4):
                    plsc.addupdate_scatter(
                        plane,
                        [jnp.full((16,), ch, _i32), yrow, zrow],
                        wvq * tk(pwv[ch]), mask=vm)

    do_chunk(jnp.int32(0), jnp.int32(128), jnp.int32(0), jnp.int32(GC),
             jnp.int32(0), jnp.int32(1))

    @pl.loop(0, 4)
    def _pp(jp):
        p = wid * 4 + jp

        @pl.loop(0, 128)
        def _zp(i):
            for ch in range(4):
                for j in range(8):
                    plane[ch, i, pl.ds(j * 16, 16)] = fzero

        plsc.subcore_barrier()

        @pl.loop(0, 4)
        def _bk(jb):
            b = (p - 2 + jb) & 127
            kx = 3 - jb
            ab = obuf[b]
            tb = tbuf[b]
            nch = (tb + (GC - 1)) // GC
            kxcol = 1 + jnp.minimum(kx, 2)

            @pl.loop(0, nch)
            def _ck(k):
                do_chunk(pl.multiple_of(ab + k * GC, 8),
                         pl.multiple_of(ab + k * GC + 128, 8),
                         k * GC, tb, kx, kxcol)

        for ch in range(4):
            pltpu.sync_copy(plane.at[ch], out_ref.at[ch, p])


def kernel(positions, particle_weights, cell):
    ns = jnp.array([NX, NY, NZ], dtype=positions.dtype)
    inv_cell = jnp.linalg.inv(cell)
    rel = ns * jnp.matmul(positions, inv_cell)

    pos_t = jnp.pad(rel.T, ((0, 0), (0, PADN - N)))
    pw_p = jnp.pad(particle_weights, ((0, PADN - N), (0, 0)))

    mesh = plsc.VectorSubcoreMesh(core_axis_name="c", subcore_axis_name="s")
    f = pl.kernel(
        _body,
        out_type=[
            jax.ShapeDtypeStruct((C, NX, NY, NZ), _f32),
            jax.ShapeDtypeStruct((2, PADN, R), _f32),
            jax.ShapeDtypeStruct((2, NPAD), _i32),
        ],
        mesh=mesh,
        scratch_types=[
            pltpu.VMEM((CHUNK,), _f32),
            pltpu.VMEM((CHUNK,), _f32),
            pltpu.VMEM((CHUNK,), _f32),
            pltpu.VMEM((CHUNK, 4), _f32),
            pltpu.VMEM((CHUNK, R), _f32),
            pltpu.VMEM((NSLOT,), _i32),
            pltpu.VMEM((2048,), _i32),
            pltpu.VMEM((16, 1024), _i32),
            pltpu.VMEM((2048,), _i32),
            pltpu.VMEM((49, 128), _i32),
            pltpu.VMEM((49, 128), _i32),
            pltpu.VMEM((C, NY, NZ), _f32),
            pltpu.VMEM((GC, R), _f32),
            pltpu.VMEM((2, 128), _i32),
            pltpu.VMEM((16, 16), _f32),
            pltpu.VMEM((16, 16), _i32),
            pltpu.VMEM((16, 16), _i32),
            pltpu.VMEM((CHUNK,), _i32),
            pltpu.VMEM_SHARED((16, 2048), _i32),
            pltpu.SMEM((136,), _i32),
            pltpu.SMEM((136,), _i32),
            pltpu.SemaphoreType.DMA,
        ],
        compiler_params=pltpu.CompilerParams(
            use_tc_tiling_on_sc=False, needs_layout_passes=False),
    )
    rho, _, _ = f(pos_t, pw_p)
    return rho


# speedup vs baseline: 28.8956x; 1.0019x over previous
import functools

import jax
import jax.numpy as jnp
from jax import lax
from jax.experimental import pallas as pl
from jax.experimental.pallas import tpu as pltpu
from jax.experimental.pallas import tpu_sc as plsc

N = 100000
NX = NY = NZ = 128
C = 4
R = 16
NPT = 6256
PADN = 100352
NSLOT = 6272
NPAD = 101632
TRASH = NPAD - 256
CHUNK = 512
TAIL = 112
GC = 256

_i32 = jnp.int32
_f32 = jnp.float32


def _p3m_weights(x):
    x2 = x * x
    x3 = x2 * x
    w0 = (1.0 - 6.0 * x + 12.0 * x2 - 8.0 * x3) * (1.0 / 48.0)
    w1 = (23.0 - 30.0 * x - 12.0 * x2 + 24.0 * x3) * (1.0 / 48.0)
    w2 = (23.0 + 30.0 * x - 12.0 * x2 - 24.0 * x3) * (1.0 / 48.0)
    w3 = (1.0 + 6.0 * x + 12.0 * x2 + 8.0 * x3) * (1.0 / 48.0)
    return w0, w1, w2, w3


def _body(pos_ref, pw_ref, out_ref, rec_ref, srt_ref,
          pxb, pyb, pzb, pwc, recbuf, ixbuf, hist, ah, cur, posb, idb,
          plane, recchunk, idchunk, z512,
          shh, obuf, tbuf, sem):
    cc_ = lax.axis_index("c")
    ss_ = lax.axis_index("s")
    iota = lax.iota(_i32, 16)
    izero = jnp.zeros((16,), _i32)
    fzero = jnp.zeros((16,), _f32)
    ione = jnp.ones((16,), _i32)

    gstart = ss_ * NPT
    np_t = jnp.where(ss_ == 15, N - 15 * NPT, NPT)


    @pl.loop(0, 128)
    def _zh(i):
        hist[pl.ds(i * 16, 16)] = izero

    @pl.loop(0, 32)
    def _zz(i):
        z512[pl.ds(i * 16, 16)] = izero

    def prep_chunk(ck, size, nv):
        gbase = pl.multiple_of(gstart + ck * CHUNK, 8)
        pltpu.sync_copy(pos_ref.at[0, pl.ds(gbase, size)], pxb.at[pl.ds(0, size)])
        pltpu.sync_copy(pos_ref.at[1, pl.ds(gbase, size)], pyb.at[pl.ds(0, size)])
        pltpu.sync_copy(pos_ref.at[2, pl.ds(gbase, size)], pzb.at[pl.ds(0, size)])
        pltpu.sync_copy(pw_ref.at[pl.ds(gbase, size)], pwc.at[pl.ds(0, size)])

        @pl.loop(0, nv)
        def _v(v):
            base = v * 16
            rows = base + iota
            rx = pxb[pl.ds(base, 16)]
            ry = pyb[pl.ds(base, 16)]
            rz = pzb[pl.ds(base, 16)]
            ixr = rx.astype(_i32)
            iyr = ry.astype(_i32)
            izr = rz.astype(_i32)
            flx = ixr.astype(_f32)
            fly = iyr.astype(_f32)
            flz = izr.astype(_f32)
            ix = ixr & 127
            iy = iyr & 127
            iz = izr & 127
            wx0, wx1, wx2, _ = _p3m_weights(rx - flx - 0.5)
            wy0, wy1, wy2, wy3 = _p3m_weights(ry - fly - 0.5)
            wz0, wz1, wz2, wz3 = _p3m_weights(rz - flz - 0.5)

            slot = ck * CHUNK + base
            valid = (slot + iota) < np_t
            plsc.addupdate_scatter(hist, [ix * 16 + iota], ione, mask=valid)
            ixbuf[pl.ds(slot, 16)] = ix

            def putf(col, val):
                plsc.store_scatter(recbuf, [rows, jnp.full((16,), col, _i32)], val)

            putf(0, plsc.bitcast(iy * 128 + iz, _f32))
            putf(1, wx0)
            putf(2, wx1)
            putf(3, wx2)
            putf(4, wy0)
            putf(5, wy1)
            putf(6, wy2)
            putf(7, wy3)
            putf(8, wz0)
            putf(9, wz1)
            putf(10, wz2)
            putf(11, wz3)
            for q in range(4):
                pv = plsc.load_gather(pwc, [rows, jnp.full((16,), q, _i32)])
                putf(12 + q, pv)

        pltpu.sync_copy(recbuf.at[pl.ds(0, size)],
                        rec_ref.at[cc_, pl.ds(gbase, size)])

    @pl.loop(0, 12)
    def _pc(ck):
        prep_chunk(ck, CHUNK, 32)

    prep_chunk(12, TAIL, 7)

    pltpu.sync_copy(hist, shh.at[ss_])

    pslice = ss_ * (NPAD // 16)

    @pl.loop(0, 12)
    def _pf(k):
        pltpu.sync_copy(z512, srt_ref.at[cc_, pl.ds(
            pl.multiple_of(pslice + k * CHUNK, 8), CHUNK)])

    pltpu.sync_copy(z512.at[pl.ds(0, NPAD // 16 - 12 * CHUNK)],
                    srt_ref.at[cc_, pl.ds(
                        pl.multiple_of(pslice + 12 * CHUNK, 8),
                        NPAD // 16 - 12 * CHUNK)])

    plsc.subcore_barrier()

    acc = jnp.int32(0)
    for h in range(2):
        pltpu.sync_copy(shh.at[:, pl.ds(h * 1024, 1024)], ah)

        def _bb(bb, a):
            b = h * 64 + bb
            run = jnp.int32(0)
            for t in range(16):
                v = ah[t, pl.ds(bb * 16, 16)]
                excl = plsc.cumsum(v) - v
                tot = jnp.sum(v)

                @pl.when(t == ss_)
                def _st():
                    cur[pl.ds(b * 16, 16)] = excl + (a + run)

                run = run + tot
            obuf[b] = a
            tbuf[b] = run
            return (a + run + 7) & ~jnp.int32(7)

        acc = pl.loop(0, 64, init_carry=acc)(_bb)

    def _rb(vv, carry):
        pcidx, ppos = carry
        base = vv * 16
        ixv = ixbuf[pl.ds(base, 16)] & 127
        valid = (base + iota) < np_t
        cidx = ixv * 16 + iota
        raw = plsc.load_gather(cur, [cidx])
        pos = jnp.where(cidx == pcidx, ppos + 1, raw)
        plsc.store_scatter(cur, [cidx], pos + 1, mask=valid)
        posv = jnp.where(valid, pos, TRASH + iota)
        idv = jnp.where(valid, gstart + base + iota, 0)
        r = vv >> 3
        col = (vv & 7) * 16 + iota
        plsc.store_scatter(posb, [jnp.full((16,), r, _i32), col], posv)
        plsc.store_scatter(idb, [jnp.full((16,), r, _i32), col], idv)
        return (jnp.where(valid, cidx, -1), pos)

    pl.loop(0, NSLOT // 16,
            init_carry=(jnp.full((16,), -1, _i32),
                        jnp.zeros((16,), _i32)))(_rb)

    @pl.loop(0, 49)
    def _sb(r):
        pltpu.sync_copy(idb.at[r], srt_ref.at[cc_].at[posb.at[r]])

    @pl.loop(0, 49)
    def _sb2(r):
        pltpu.sync_copy(srt_ref.at[cc_].at[posb.at[r]], idb.at[r])

    plsc.subcore_barrier()
    @pl.loop(0, 12)
    def _fl(k):
        pltpu.sync_copy(srt_ref.at[cc_, pl.ds(
            pl.multiple_of(pslice + k * CHUNK, 8), CHUNK)], z512)

    pltpu.sync_copy(srt_ref.at[cc_, pl.ds(
        pl.multiple_of(pslice + 12 * CHUNK, 8),
        NPAD // 16 - 12 * CHUNK)],
        z512.at[pl.ds(0, NPAD // 16 - 12 * CHUNK)])

    plsc.subcore_barrier()

    wid = ss_ * 2 + cc_

    def do_chunk(st, st1, cnt0, tb, kx, kxcol):
        pltpu.sync_copy(srt_ref.at[cc_, pl.ds(st, 128)], idchunk.at[0])
        pltpu.sync_copy(srt_ref.at[cc_, pl.ds(st1, 128)], idchunk.at[1])
        d0 = pltpu.async_copy(rec_ref.at[cc_].at[idchunk.at[0]],
                              recchunk.at[pl.ds(0, 128)], sem)
        d1 = pltpu.async_copy(rec_ref.at[cc_].at[idchunk.at[1]],
                              recchunk.at[pl.ds(128, 128)], sem)
        d0.wait()
        d1.wait()

        dym = (iota >> 2) - 1
        dzm = (iota & 3) - 1
        dyv = iota >> 2
        dzv = iota & 3

        @pl.loop(0, 16)
        def _v(v):
            rows = v * 16 + iota

            def g(col):
                return plsc.load_gather(
                    recchunk, [rows, jnp.full((16,), col, _i32)])

            yz = plsc.bitcast(g(0), _i32)
            iy = yz >> 7
            iz = yz & 127
            w0, w1, w2 = g(1), g(2), g(3)
            wa = plsc.load_gather(
                recchunk, [rows, jnp.full((16,), kxcol, _i32)])
            wxk = jnp.where(kx == 3, 1.0 - w0 - w1 - w2, wa)
            wy = [g(4 + d) for d in range(4)]
            wz = [g(8 + d) for d in range(4)]
            pwv = [g(12 + q) for q in range(4)]
            cnt = cnt0 + v * 16
            _dn = lax.GatherDimensionNumbers(
                offset_dims=(), collapsed_slice_dims=(0,),
                start_index_map=(0,))
            for q in range(16):
                fq = jnp.full((16, 1), q, _i32)

                def tk(x):
                    return lax.gather(
                        x, fq, dimension_numbers=_dn, slice_sizes=(1,),
                        mode=lax.GatherScatterMode.PROMISE_IN_BOUNDS)

                yrow = (tk(iy) + dym) & 127
                zrow = (tk(iz) + dzm) & 127
                wyv = jnp.where(
                    dyv == 0, tk(wy[0]),
                    jnp.where(dyv == 1, tk(wy[1]),
                              jnp.where(dyv == 2, tk(wy[2]), tk(wy[3]))))
                wzv = jnp.where(
                    dzv == 0, tk(wz[0]),
                    jnp.where(dzv == 1, tk(wz[1]),
                              jnp.where(dzv == 2, tk(wz[2]), tk(wz[3]))))
                wvq = tk(wxk) * wyv * wzv
                vm = jnp.full((16,), (cnt + q) < tb)
                for ch in range(4):
                    plsc.addupdate_scatter(
                        plane,
                        [jnp.full((16,), ch, _i32), yrow, zrow],
                        wvq * tk(pwv[ch]), mask=vm)

    do_chunk(jnp.int32(0), jnp.int32(128), jnp.int32(0), jnp.int32(GC),
             jnp.int32(0), jnp.int32(1))

    @pl.loop(0, 4)
    def _pp(jp):
        p = wid * 4 + jp

        @pl.loop(0, 128)
        def _zp(i):
            for ch in range(4):
                for j in range(8):
                    plane[ch, i, pl.ds(j * 16, 16)] = fzero

        plsc.subcore_barrier()

        @pl.loop(0, 4)
        def _bk(jb):
            b = (p - 2 + jb) & 127
            kx = 3 - jb
            ab = obuf[b]
            tb = tbuf[b]
            nch = (tb + (GC - 1)) // GC
            kxcol = 1 + jnp.minimum(kx, 2)

            @pl.loop(0, nch)
            def _ck(k):
                do_chunk(pl.multiple_of(ab + k * GC, 8),
                         pl.multiple_of(ab + k * GC + 128, 8),
                         k * GC, tb, kx, kxcol)

        for ch in range(4):
            pltpu.sync_copy(plane.at[ch], out_ref.at[ch, p])


def kernel(positions, particle_weights, cell):
    ns = jnp.array([NX, NY, NZ], dtype=positions.dtype)
    inv_cell = jnp.linalg.inv(cell)
    rel = ns * jnp.matmul(positions, inv_cell)

    pos_t = jnp.pad(rel.T, ((0, 0), (0, PADN - N)))
    pw_p = jnp.pad(particle_weights, ((0, PADN - N), (0, 0)))

    mesh = plsc.VectorSubcoreMesh(core_axis_name="c", subcore_axis_name="s")
    f = pl.kernel(
        _body,
        out_type=[
            jax.ShapeDtypeStruct((C, NX, NY, NZ), _f32),
            jax.ShapeDtypeStruct((2, PADN, R), _f32),
            jax.ShapeDtypeStruct((2, NPAD), _i32),
        ],
        mesh=mesh,
        scratch_types=[
            pltpu.VMEM((CHUNK,), _f32),
            pltpu.VMEM((CHUNK,), _f32),
            pltpu.VMEM((CHUNK,), _f32),
            pltpu.VMEM((CHUNK, 4), _f32),
            pltpu.VMEM((CHUNK, R), _f32),
            pltpu.VMEM((NSLOT,), _i32),
            pltpu.VMEM((2048,), _i32),
            pltpu.VMEM((16, 1024), _i32),
            pltpu.VMEM((2048,), _i32),
            pltpu.VMEM((49, 128), _i32),
            pltpu.VMEM((49, 128), _i32),
            pltpu.VMEM((C, NY, NZ), _f32),
            pltpu.VMEM((GC, R), _f32),
            pltpu.VMEM((2, 128), _i32),
            pltpu.VMEM((CHUNK,), _i32),
            pltpu.VMEM_SHARED((16, 2048), _i32),
            pltpu.SMEM((136,), _i32),
            pltpu.SMEM((136,), _i32),
            pltpu.SemaphoreType.DMA,
        ],
        compiler_params=pltpu.CompilerParams(
            use_tc_tiling_on_sc=False, needs_layout_passes=False),
    )
    rho, _, _ = f(pos_t, pw_p)
    return rho
